# baseline (speedup 1.0000x reference)
import functools

import jax
import jax.numpy as jnp
from jax import lax
from jax.experimental import pallas as pl
from jax.experimental.pallas import tpu as pltpu
from jax.experimental.pallas import tpu_sc as plsc

N = 10000
E = 320000
D_IN = 128
D_H = 16
D_OUT = 128

NW = 32
IB = 128
SUB = 1
NJ = 80
PW = NJ * SUB * IB
EPAD = NW * PW
NPAD = 10112
RPT = NPAD // 16
NBUF = 12
LOOKAHEAD = 6

_mesh = plsc.VectorSubcoreMesh(core_axis_name="c", subcore_axis_name="s")


@functools.partial(
    pl.kernel,
    out_type=jax.ShapeDtypeStruct((2 * NPAD,), jnp.float32),
    mesh=_mesh,
    compiler_params=pltpu.CompilerParams(use_tc_tiling_on_sc=False),
    scratch_types=[
        pltpu.VMEM((NJ, SUB * IB), jnp.int32),
        pltpu.VMEM((SUB * IB,), jnp.float32),
        pltpu.VMEM((RPT,), jnp.float32),
        pltpu.VMEM_SHARED((NPAD,), jnp.float32),
    ],
)
def _sc_deg(col_hbm, zero_hbm, out_hbm, colv, ones_v, zb, acc):
    c = lax.axis_index("c")
    s = lax.axis_index("s")
    w = s * 2 + c
    pltpu.sync_copy(col_hbm.at[w], colv)
    for i in range(SUB * IB // 16):
        ones_v[pl.ds(i * 16, 16)] = jnp.full((16,), 1.0, jnp.float32)
    r0 = s * RPT
    pltpu.sync_copy(zero_hbm.at[pl.ds(r0, RPT)], zb)
    pltpu.sync_copy(zb, acc.at[pl.ds(r0, RPT)])
    plsc.subcore_barrier()

    def body(j, carry):
        pltpu.sync_copy(ones_v, acc.at[colv.at[j]], add=True)
        return carry

    lax.fori_loop(0, NJ, body, 0)
    plsc.subcore_barrier()
    pltpu.sync_copy(acc.at[pl.ds(r0, RPT)], zb)
    pltpu.sync_copy(zb, out_hbm.at[pl.ds(c * NPAD + r0, RPT)])


@functools.partial(
    pl.kernel,
    out_type=jax.ShapeDtypeStruct((2 * NPAD, D_H), jnp.float32),
    mesh=_mesh,
    compiler_params=pltpu.CompilerParams(use_tc_tiling_on_sc=False),
    scratch_types=[
        pltpu.VMEM((NJ, SUB * IB), jnp.int32),
        pltpu.VMEM((NJ, SUB * IB), jnp.int32),
        pltpu.VMEM((NBUF, SUB * IB, D_H), jnp.float32),
        pltpu.VMEM((RPT, D_H), jnp.float32),
        pltpu.VMEM_SHARED((NPAD, D_H), jnp.float32),
        pltpu.SemaphoreType.DMA((NBUF,)),
        pltpu.SemaphoreType.DMA((NBUF,)),
    ],
)
def _sc_agg(g_hbm, row_hbm, col_hbm, zero_hbm, out_hbm,
            rowv, colv, buf, zb, acc, gsem, ssem):
    c = lax.axis_index("c")
    s = lax.axis_index("s")
    w = s * 2 + c
    pltpu.sync_copy(row_hbm.at[w], rowv)
    pltpu.sync_copy(col_hbm.at[w], colv)
    r0 = s * RPT
    pltpu.sync_copy(zero_hbm.at[pl.ds(r0, RPT)], zb)
    pltpu.sync_copy(zb, acc.at[pl.ds(r0, RPT)])
    plsc.subcore_barrier()

    for b in range(LOOKAHEAD):
        pltpu.async_copy(g_hbm.at[rowv.at[b]], buf.at[b], gsem.at[b])

    def body(j, carry):
        p = lax.rem(j, NBUF)
        pltpu.make_async_copy(g_hbm.at[rowv.at[j]], buf.at[p], gsem.at[p]).wait()
        pltpu.async_copy(buf.at[p], acc.at[colv.at[j]], ssem.at[p], add=True)

        @pl.when(j >= LOOKAHEAD)
        def _drain_old_scatter():
            q = lax.rem(j - LOOKAHEAD, NBUF)
            pltpu.make_async_copy(
                buf.at[q], acc.at[colv.at[j - LOOKAHEAD]], ssem.at[q]
            ).wait()

        @pl.when(j + LOOKAHEAD < NJ)
        def _issue_next_gather():
            q = lax.rem(j + LOOKAHEAD, NBUF)
            pltpu.async_copy(
                g_hbm.at[rowv.at[j + LOOKAHEAD]], buf.at[q], gsem.at[q]
            )

        return carry

    lax.fori_loop(0, NJ, body, 0)

    def drain(j, carry):
        q = lax.rem(j, NBUF)
        pltpu.make_async_copy(buf.at[q], acc.at[colv.at[j]], ssem.at[q]).wait()
        return carry

    lax.fori_loop(NJ - LOOKAHEAD, NJ, drain, 0)
    plsc.subcore_barrier()
    pltpu.sync_copy(acc.at[pl.ds(r0, RPT)], zb)
    pltpu.sync_copy(zb, out_hbm.at[pl.ds(c * NPAD + r0, RPT)])


def _tc_mm1(xp, W1):
    def body(x_ref, w_ref, h_ref):
        h_ref[...] = jnp.dot(
            x_ref[...], w_ref[...], preferred_element_type=jnp.float32
        )

    return pl.pallas_call(
        body,
        out_shape=jax.ShapeDtypeStruct((NPAD, D_H), jnp.float32),
    )(xp, W1)


def _tc_scale1(h1, deg_t):
    def body(h_ref, dp_ref, g_ref, dinv_ref):
        dp = dp_ref[...]
        deg = dp[:, 0:1] + dp[:, 1:2] + 1.0
        dinv = lax.rsqrt(deg)
        g_ref[...] = h_ref[...] * dinv
        dinv_ref[...] = dinv

    return pl.pallas_call(
        body,
        out_shape=[
            jax.ShapeDtypeStruct((NPAD, D_H), jnp.float32),
            jax.ShapeDtypeStruct((NPAD, 1), jnp.float32),
        ],
    )(h1, deg_t)


def _tc_layer2_prep(t1p, g1, dinv, b1):
    def body(t_ref, g_ref, d_ref, b_ref, r_ref):
        sums = t_ref[0] + t_ref[1] + g_ref[...]
        dinv = d_ref[...]
        pre = dinv * sums + b_ref[...]
        r = jnp.maximum(pre, 0.0) * dinv
        rowid = lax.broadcasted_iota(jnp.int32, (NPAD, D_H), 0)
        r_ref[...] = jnp.where(rowid < N, r, 0.0)

    return pl.pallas_call(
        body,
        out_shape=jax.ShapeDtypeStruct((NPAD, D_H), jnp.float32),
    )(t1p, g1, dinv, b1)


def _tc_output(t2p, r, dinv, W2, b2):
    def body(t_ref, r_ref, d_ref, w_ref, b_ref, o_ref):
        t = d_ref[...] * (t_ref[0] + t_ref[1] + r_ref[...])
        o_ref[...] = (
            jnp.dot(t, w_ref[...], preferred_element_type=jnp.float32)
            + b_ref[...]
        )

    return pl.pallas_call(
        body,
        out_shape=jax.ShapeDtypeStruct((NPAD, D_OUT), jnp.float32),
    )(t2p, r, dinv, W2, b2)


def kernel(x, edge_index, W1, b1, W2, b2):
    x = x.astype(jnp.float32)
    ei = edge_index.astype(jnp.int32)
    pad_idx = N + jnp.arange(EPAD, dtype=jnp.int32) % (NPAD - N)
    rowp = pad_idx.at[:E].set(ei[0]).reshape(NW, NJ, SUB * IB)
    colp = pad_idx.at[:E].set(ei[1]).reshape(NW, NJ, SUB * IB)
    xp = jnp.pad(x, ((0, NPAD - N), (0, 0)))
    z1 = jnp.zeros((NPAD,), jnp.float32)
    z16 = jnp.zeros((NPAD, D_H), jnp.float32)

    h1 = _tc_mm1(xp, W1)
    degp = _sc_deg(colp, z1).reshape(2, NPAD)
    g1, dinv = _tc_scale1(h1, degp.T)
    t1p = _sc_agg(g1, rowp, colp, z16).reshape(2, NPAD, D_H)
    r = _tc_layer2_prep(t1p, g1, dinv, b1.reshape(1, D_H))
    t2p = _sc_agg(r, rowp, colp, z16).reshape(2, NPAD, D_H)
    outp = _tc_output(t2p, r, dinv, W2, b2.reshape(1, D_OUT))
    return outp[:N]


# speedup vs baseline: 1.1057x; 1.1057x over previous
import functools

import jax
import jax.numpy as jnp
from jax import lax
from jax.experimental import pallas as pl
from jax.experimental.pallas import tpu as pltpu
from jax.experimental.pallas import tpu_sc as plsc

N = 10000
E = 320000
D_IN = 128
D_H = 16
D_OUT = 128

NW = 32
IB = 128
SUB = 1
NJ = 80
PW = NJ * SUB * IB
EPAD = NW * PW
NPAD = 10112
RPT = NPAD // 16
NBUF = 12
LOOKAHEAD = 6

_mesh = plsc.VectorSubcoreMesh(core_axis_name="c", subcore_axis_name="s")


@functools.partial(
    pl.kernel,
    out_type=jax.ShapeDtypeStruct((2 * NPAD,), jnp.float32),
    mesh=_mesh,
    compiler_params=pltpu.CompilerParams(use_tc_tiling_on_sc=False),
    scratch_types=[
        pltpu.VMEM((NJ, SUB * IB), jnp.int32),
        pltpu.VMEM((SUB * IB,), jnp.float32),
        pltpu.VMEM((RPT,), jnp.float32),
        pltpu.VMEM_SHARED((NPAD,), jnp.float32),
    ],
)
def _sc_deg(col_hbm, zero_hbm, out_hbm, colv, ones_v, zb, acc):
    c = lax.axis_index("c")
    s = lax.axis_index("s")
    w = s * 2 + c
    pltpu.sync_copy(col_hbm.at[w], colv)
    for i in range(SUB * IB // 16):
        ones_v[pl.ds(i * 16, 16)] = jnp.full((16,), 1.0, jnp.float32)
    r0 = s * RPT
    pltpu.sync_copy(zero_hbm.at[pl.ds(r0, RPT)], zb)
    pltpu.sync_copy(zb, acc.at[pl.ds(r0, RPT)])
    plsc.subcore_barrier()

    def body(j, carry):
        pltpu.sync_copy(ones_v, acc.at[colv.at[j]], add=True)
        return carry

    lax.fori_loop(0, NJ, body, 0)
    plsc.subcore_barrier()
    pltpu.sync_copy(acc.at[pl.ds(r0, RPT)], zb)
    pltpu.sync_copy(zb, out_hbm.at[pl.ds(c * NPAD + r0, RPT)])


@functools.partial(
    pl.kernel,
    out_type=jax.ShapeDtypeStruct((2 * NPAD, D_H), jnp.float32),
    mesh=_mesh,
    compiler_params=pltpu.CompilerParams(use_tc_tiling_on_sc=False),
    scratch_types=[
        pltpu.VMEM((NJ, SUB * IB), jnp.int32),
        pltpu.VMEM((NJ, SUB * IB), jnp.int32),
        pltpu.VMEM((NBUF, SUB * IB, D_H), jnp.float32),
        pltpu.VMEM((RPT, D_H), jnp.float32),
        pltpu.VMEM_SHARED((NPAD, D_H), jnp.float32),
        pltpu.SemaphoreType.DMA((NBUF,)),
        pltpu.SemaphoreType.DMA((NBUF,)),
    ],
)
def _sc_agg(g_hbm, row_hbm, col_hbm, zero_hbm, out_hbm,
            rowv, colv, buf, zb, acc, gsem, ssem):
    c = lax.axis_index("c")
    s = lax.axis_index("s")
    w = s * 2 + c
    pltpu.sync_copy(row_hbm.at[w], rowv)
    pltpu.sync_copy(col_hbm.at[w], colv)
    r0 = s * RPT
    pltpu.sync_copy(zero_hbm.at[pl.ds(r0, RPT)], zb)
    pltpu.sync_copy(zb, acc.at[pl.ds(r0, RPT)])
    plsc.subcore_barrier()

    for b in range(LOOKAHEAD):
        pltpu.async_copy(g_hbm.at[rowv.at[b]], buf.at[b], gsem.at[b])

    def body(j, carry):
        p = lax.rem(j, NBUF)
        pltpu.make_async_copy(g_hbm.at[rowv.at[j]], buf.at[p], gsem.at[p]).wait()
        pltpu.async_copy(buf.at[p], acc.at[colv.at[j]], ssem.at[p], add=True)

        @pl.when(j >= LOOKAHEAD)
        def _drain_old_scatter():
            q = lax.rem(j - LOOKAHEAD, NBUF)
            pltpu.make_async_copy(
                buf.at[q], acc.at[colv.at[j - LOOKAHEAD]], ssem.at[q]
            ).wait()

        @pl.when(j + LOOKAHEAD < NJ)
        def _issue_next_gather():
            q = lax.rem(j + LOOKAHEAD, NBUF)
            pltpu.async_copy(
                g_hbm.at[rowv.at[j + LOOKAHEAD]], buf.at[q], gsem.at[q]
            )

        return carry

    lax.fori_loop(0, NJ, body, 0)

    def drain(j, carry):
        q = lax.rem(j, NBUF)
        pltpu.make_async_copy(buf.at[q], acc.at[colv.at[j]], ssem.at[q]).wait()
        return carry

    lax.fori_loop(NJ - LOOKAHEAD, NJ, drain, 0)
    plsc.subcore_barrier()
    pltpu.sync_copy(acc.at[pl.ds(r0, RPT)], zb)
    pltpu.sync_copy(zb, out_hbm.at[pl.ds(c * NPAD + r0, RPT)])


@functools.partial(
    pl.kernel,
    out_type=[
        jax.ShapeDtypeStruct((2 * NPAD, D_H), jnp.float32),
        jax.ShapeDtypeStruct((NPAD, D_H), jnp.float32),
    ],
    mesh=_mesh,
    compiler_params=pltpu.CompilerParams(use_tc_tiling_on_sc=False),
    scratch_types=[
        pltpu.VMEM((NJ, SUB * IB), jnp.int32),
        pltpu.VMEM((NJ, SUB * IB), jnp.int32),
        pltpu.VMEM((NBUF, SUB * IB, D_H), jnp.float32),
        pltpu.VMEM((RPT, D_H), jnp.float32),
        pltpu.VMEM((RPT, D_H), jnp.float32),
        pltpu.VMEM((RPT, D_H), jnp.float32),
        pltpu.VMEM((RPT, D_H), jnp.float32),
        pltpu.VMEM((RPT, D_H), jnp.float32),
        pltpu.VMEM((16,), jnp.float32),
        pltpu.VMEM_SHARED((NPAD, D_H), jnp.float32),
        pltpu.VMEM_SHARED((NPAD, D_H), jnp.float32),
        pltpu.SemaphoreType.DMA((NBUF,)),
        pltpu.SemaphoreType.DMA((NBUF,)),
    ],
)
def _sc_agg2(t1p_hbm, g1_hbm, dinv_hbm, b1_hbm, row_hbm, col_hbm, zero_hbm,
             t2_hbm, r_hbm,
             rowv, colv, buf, zb, t0b, t1b, gb, dvb, b1v, acc, r_sp,
             gsem, ssem):
    c = lax.axis_index("c")
    s = lax.axis_index("s")
    w = s * 2 + c
    pltpu.sync_copy(row_hbm.at[w], rowv)
    pltpu.sync_copy(col_hbm.at[w], colv)
    r0 = s * RPT
    pltpu.sync_copy(zero_hbm.at[pl.ds(r0, RPT)], zb)
    pltpu.sync_copy(zb, acc.at[pl.ds(r0, RPT)])
    pltpu.sync_copy(t1p_hbm.at[pl.ds(r0, RPT)], t0b)
    pltpu.sync_copy(t1p_hbm.at[pl.ds(NPAD + r0, RPT)], t1b)
    pltpu.sync_copy(g1_hbm.at[pl.ds(r0, RPT)], gb)
    pltpu.sync_copy(dinv_hbm.at[pl.ds(r0, RPT)], dvb)
    pltpu.sync_copy(b1_hbm, b1v)

    def erow(i, carry):
        dv = dvb[i]
        pre = (t0b[i] + t1b[i] + gb[i]) * dv + b1v[...]

        rrow = jnp.maximum(pre, 0.0) * dv
        keep = (r0 + i) < N
        gb[i] = jnp.where(keep, rrow, jnp.zeros((D_H,), jnp.float32))
        return carry

    lax.fori_loop(0, RPT, erow, 0)
    pltpu.sync_copy(gb, r_sp.at[pl.ds(r0, RPT)])

    @pl.when(c == 0)
    def _publish_r():
        pltpu.sync_copy(gb, r_hbm.at[pl.ds(r0, RPT)])

    plsc.subcore_barrier()

    for b in range(LOOKAHEAD):
        pltpu.async_copy(r_sp.at[rowv.at[b]], buf.at[b], gsem.at[b])

    def body(j, carry):
        p = lax.rem(j, NBUF)
        pltpu.make_async_copy(r_sp.at[rowv.at[j]], buf.at[p], gsem.at[p]).wait()
        pltpu.async_copy(buf.at[p], acc.at[colv.at[j]], ssem.at[p], add=True)

        @pl.when(j >= LOOKAHEAD)
        def _drain_old_scatter():
            q = lax.rem(j - LOOKAHEAD, NBUF)
            pltpu.make_async_copy(
                buf.at[q], acc.at[colv.at[j - LOOKAHEAD]], ssem.at[q]
            ).wait()

        @pl.when(j + LOOKAHEAD < NJ)
        def _issue_next_gather():
            q = lax.rem(j + LOOKAHEAD, NBUF)
            pltpu.async_copy(
                r_sp.at[rowv.at[j + LOOKAHEAD]], buf.at[q], gsem.at[q]
            )

        return carry

    lax.fori_loop(0, NJ, body, 0)

    def drain(j, carry):
        q = lax.rem(j, NBUF)
        pltpu.make_async_copy(buf.at[q], acc.at[colv.at[j]], ssem.at[q]).wait()
        return carry

    lax.fori_loop(NJ - LOOKAHEAD, NJ, drain, 0)
    plsc.subcore_barrier()
    pltpu.sync_copy(acc.at[pl.ds(r0, RPT)], zb)
    pltpu.sync_copy(zb, t2_hbm.at[pl.ds(c * NPAD + r0, RPT)])


def _tc_layer1(xp, W1, deg_t):
    def body(x_ref, w_ref, dp_ref, g_ref, dinv_ref):
        dp = dp_ref[...]
        deg = dp[:, 0:1] + dp[:, 1:2] + 1.0
        dinv = lax.rsqrt(deg)
        h = jnp.dot(x_ref[...], w_ref[...], preferred_element_type=jnp.float32)
        g_ref[...] = h * dinv
        dinv_ref[...] = jnp.broadcast_to(dinv, (NPAD, D_H))

    return pl.pallas_call(
        body,
        out_shape=[
            jax.ShapeDtypeStruct((NPAD, D_H), jnp.float32),
            jax.ShapeDtypeStruct((NPAD, D_H), jnp.float32),
        ],
    )(xp, W1, deg_t)


def _tc_layer2_prep(t1p, g1, dinv, b1):
    def body(t_ref, g_ref, d_ref, b_ref, r_ref):
        sums = t_ref[0] + t_ref[1] + g_ref[...]
        dinv = d_ref[...]
        pre = dinv * sums + b_ref[...]
        r = jnp.maximum(pre, 0.0) * dinv
        rowid = lax.broadcasted_iota(jnp.int32, (NPAD, D_H), 0)
        r_ref[...] = jnp.where(rowid < N, r, 0.0)

    return pl.pallas_call(
        body,
        out_shape=jax.ShapeDtypeStruct((NPAD, D_H), jnp.float32),
    )(t1p, g1, dinv, b1)


def _tc_output(t2p, r, dinv, W2, b2):
    def body(t_ref, r_ref, d_ref, w_ref, b_ref, o_ref):
        t = d_ref[...] * (t_ref[0] + t_ref[1] + r_ref[...])
        o_ref[...] = (
            jnp.dot(t, w_ref[...], preferred_element_type=jnp.float32)
            + b_ref[...]
        )

    return pl.pallas_call(
        body,
        out_shape=jax.ShapeDtypeStruct((NPAD, D_OUT), jnp.float32),
    )(t2p, r, dinv, W2, b2)


def kernel(x, edge_index, W1, b1, W2, b2):
    x = x.astype(jnp.float32)
    ei = edge_index.astype(jnp.int32)
    pad_idx = N + jnp.arange(EPAD, dtype=jnp.int32) % (NPAD - N)
    rowp = pad_idx.at[:E].set(ei[0]).reshape(NW, NJ, SUB * IB)
    colp = pad_idx.at[:E].set(ei[1]).reshape(NW, NJ, SUB * IB)
    xp = jnp.pad(x, ((0, NPAD - N), (0, 0)))
    z1 = jnp.zeros((NPAD,), jnp.float32)
    z16 = jnp.zeros((NPAD, D_H), jnp.float32)

    degp = _sc_deg(colp, z1).reshape(2, NPAD)
    g1, dinv = _tc_layer1(xp, W1, degp.T)
    t1p = _sc_agg(g1, rowp, colp, z16)
    t2p, r = _sc_agg2(t1p, g1, dinv, b1.astype(jnp.float32),
                      rowp, colp, z16)
    outp = _tc_output(t2p.reshape(2, NPAD, D_H), r, dinv, W2,
                      b2.reshape(1, D_OUT))
    return outp[:N]
